# Initial kernel scaffold; baseline (speedup 1.0000x reference)
import functools

import jax
import jax.numpy as jnp
from jax import lax
from jax.experimental import pallas as pl
from jax.experimental.pallas import tpu as pltpu
from jax.experimental.pallas import tpu_sc as plsc

VOCAB = 1_000_000
EMBED_DIM = 32
BATCH = 4096
SEQ_LEN = 200

_NC = 2
_NS = 16
_NW = _NC * _NS
_B = BATCH * SEQ_LEN
_BPW = _B // _NW
_C = 128
_NCHUNK = _BPW // _C


def _make_sc_gather():
  mesh = plsc.VectorSubcoreMesh(core_axis_name="c", subcore_axis_name="s")

  @functools.partial(
      pl.kernel,
      out_type=jax.ShapeDtypeStruct((_NW, _NCHUNK, _C, EMBED_DIM), jnp.float32),
      mesh=mesh,
      scratch_types=[
          pltpu.VMEM((_NCHUNK, _C), jnp.int32),
          pltpu.VMEM((2, _C, EMBED_DIM), jnp.float32),
          pltpu.SemaphoreType.DMA,
          pltpu.SemaphoreType.DMA,
          pltpu.SemaphoreType.DMA,
          pltpu.SemaphoreType.DMA,
      ],
  )
  def sc_gather(x_hbm, w_hbm, out_hbm, idx_v, buf_v, g0, g1, s0, s1):
    wid = lax.axis_index("s") * _NC + lax.axis_index("c")
    pltpu.sync_copy(x_hbm.at[wid], idx_v)

    gsems = (g0, g1)
    ssems = (s0, s1)

    pltpu.async_copy(w_hbm.at[idx_v.at[0]], buf_v.at[0], g0)
    pltpu.async_copy(w_hbm.at[idx_v.at[1]], buf_v.at[1], g1)

    @pl.loop(0, _NCHUNK, step=2)
    def _(j):
      for b in range(2):
        jb = j + b
        pltpu.make_async_copy(
            w_hbm.at[idx_v.at[0]], buf_v.at[b], gsems[b]).wait()
        pltpu.async_copy(buf_v.at[b], out_hbm.at[wid, jb], ssems[b])
      for b in range(2):
        jb = j + b

        @pl.when(jb + 2 < _NCHUNK)
        def _():
          pltpu.make_async_copy(
              buf_v.at[b], out_hbm.at[wid, jb], ssems[b]).wait()
          pltpu.async_copy(w_hbm.at[idx_v.at[jb + 2]], buf_v.at[b], gsems[b])

    for b in range(2):
      pltpu.make_async_copy(
          buf_v.at[b], out_hbm.at[wid, _NCHUNK - 2 + b], ssems[b]).wait()

  return sc_gather


_sc_gather = _make_sc_gather()


@jax.jit
def kernel(x, w):
  xi = x.reshape(_NW, _NCHUNK, _C).astype(jnp.int32)
  out = _sc_gather(xi, w)
  return out.reshape(BATCH, SEQ_LEN, EMBED_DIM)


# speedup vs baseline: 1.4188x; 1.4188x over previous
import functools

import jax
import jax.numpy as jnp
from jax import lax
from jax.experimental import pallas as pl
from jax.experimental.pallas import tpu as pltpu
from jax.experimental.pallas import tpu_sc as plsc

VOCAB = 1_000_000
EMBED_DIM = 32
BATCH = 4096
SEQ_LEN = 200

_NC = 2
_NS = 16
_NW = _NC * _NS
_B = BATCH * SEQ_LEN
_BPW = _B // _NW
_C = 128
_NCHUNK = _BPW // _C


def _make_sc_gather():
  mesh = plsc.VectorSubcoreMesh(core_axis_name="c", subcore_axis_name="s")

  @functools.partial(
      pl.kernel,
      out_type=jax.ShapeDtypeStruct((_NW, _NCHUNK, _C, EMBED_DIM), jnp.float32),
      mesh=mesh,
      compiler_params=pltpu.CompilerParams(use_tc_tiling_on_sc=False),
      scratch_types=[
          pltpu.VMEM((_NCHUNK, _C), jnp.int32),
          pltpu.VMEM((2, _C, EMBED_DIM), jnp.float32),
          pltpu.SemaphoreType.DMA,
          pltpu.SemaphoreType.DMA,
          pltpu.SemaphoreType.DMA,
          pltpu.SemaphoreType.DMA,
      ],
  )
  def sc_gather(x_hbm, w_hbm, out_hbm, idx_v, buf_v, g0, g1, s0, s1):
    wid = lax.axis_index("s") * _NC + lax.axis_index("c")
    pltpu.sync_copy(x_hbm.at[wid], idx_v)

    gsems = (g0, g1)
    ssems = (s0, s1)

    pltpu.async_copy(w_hbm.at[idx_v.at[0]], buf_v.at[0], g0)
    pltpu.async_copy(w_hbm.at[idx_v.at[1]], buf_v.at[1], g1)

    @pl.loop(0, _NCHUNK, step=2)
    def _(j):
      for b in range(2):
        jb = j + b
        pltpu.make_async_copy(
            w_hbm.at[idx_v.at[0]], buf_v.at[b], gsems[b]).wait()
        pltpu.async_copy(buf_v.at[b], out_hbm.at[wid, jb], ssems[b])
      for b in range(2):
        jb = j + b

        @pl.when(jb + 2 < _NCHUNK)
        def _():
          pltpu.make_async_copy(
              buf_v.at[b], out_hbm.at[wid, jb], ssems[b]).wait()
          pltpu.async_copy(w_hbm.at[idx_v.at[jb + 2]], buf_v.at[b], gsems[b])

    for b in range(2):
      pltpu.make_async_copy(
          buf_v.at[b], out_hbm.at[wid, _NCHUNK - 2 + b], ssems[b]).wait()

  return sc_gather


_sc_gather = _make_sc_gather()


@jax.jit
def kernel(x, w):
  xi = x.reshape(_NW, _NCHUNK, _C).astype(jnp.int32)
  out = _sc_gather(xi, w)
  return out.reshape(BATCH, SEQ_LEN, EMBED_DIM)
